# baseline (speedup 1.0000x reference)
import functools

import jax
import jax.numpy as jnp
from jax import lax
from jax.experimental import pallas as pl
from jax.experimental.pallas import tpu as pltpu
from jax.experimental.pallas import tpu_sc as plsc

L = 16
NC = 2
NS = 16
NW = NC * NS
NBUF = 3


def _pe_add_kernel(N, D, rows_per_w, R):
    n_chunks = rows_per_w // R
    mesh = plsc.VectorSubcoreMesh(core_axis_name="c", subcore_axis_name="s")

    buf_types = [pltpu.VMEM((R, D), jnp.float32) for _ in range(2 * NBUF)]
    sem_types = [pltpu.SemaphoreType.DMA for _ in range(3 * NBUF)]

    @functools.partial(
        pl.kernel,
        out_type=jax.ShapeDtypeStruct((N, D), jnp.float32),
        mesh=mesh,
        scratch_types=[
            pltpu.VMEM((rows_per_w,), jnp.int32),
            pltpu.VMEM((rows_per_w + L,), jnp.float32),
        ] + buf_types + sem_types,
    )
    def k(x_hbm, pos_hbm, mask_hbm, pe_hbm, out_hbm, idx_v, mask_v, *rest):
        xbufs = rest[0:NBUF]
        pbufs = rest[NBUF:2 * NBUF]
        sems_x = rest[2 * NBUF:2 * NBUF + NBUF]
        sems_p = rest[3 * NBUF:3 * NBUF + NBUF]
        sems_o = rest[4 * NBUF:4 * NBUF + NBUF]

        wid = lax.axis_index("s") * NC + lax.axis_index("c")
        base = wid * rows_per_w
        pltpu.sync_copy(pos_hbm.at[pl.ds(base, rows_per_w)], idx_v)
        pltpu.sync_copy(mask_hbm.at[pl.ds(base, rows_per_w)],
                        mask_v.at[pl.ds(0, rows_per_w)])

        def min_body(i, acc):
            return jnp.minimum(acc, mask_v[pl.ds(i * L, L)])

        acc = lax.fori_loop(0, rows_per_w // L, min_body,
                            jnp.full((L,), 1.0, jnp.float32))
        m_min = acc[0]
        for i in range(1, L):
            m_min = jnp.minimum(m_min, acc[i])

        def start_loads(c, slot):
            cx = pltpu.async_copy(
                x_hbm.at[pl.ds(base + c * R, R)], xbufs[slot], sems_x[slot])
            cp = pltpu.async_copy(
                pe_hbm.at[idx_v.at[pl.ds(c * R, R)]], pbufs[slot],
                sems_p[slot])
            return cx, cp

        @pl.when(m_min > 0.5)
        def _fast():
            cp_x = [None] * NBUF
            cp_p = [None] * NBUF
            cp_o = [None] * NBUF
            for c in range(min(2, n_chunks)):
                cp_x[c], cp_p[c] = start_loads(c, c)
            for c in range(n_chunks):
                s = c % NBUF
                if c + 2 < n_chunks:
                    s2 = (c + 2) % NBUF
                    if cp_o[s2] is not None:
                        cp_o[s2].wait()
                    cp_x[s2], cp_p[s2] = start_loads(c + 2, s2)
                cp_x[s].wait()
                cp_p[s].wait()
                xb = xbufs[s]
                pb = pbufs[s]

                def combine(r, carry):
                    for j in range(D // L):
                        sl = pl.ds(j * L, L)
                        plsc.addupdate(xb.at[r, sl], pb[r, sl])
                    return carry

                lax.fori_loop(0, R, combine, 0)

                cp_o[s] = pltpu.async_copy(
                    xb, out_hbm.at[pl.ds(base + c * R, R)], sems_o[s])
            for s in range(NBUF):
                if cp_o[s] is not None:
                    cp_o[s].wait()

        @pl.when(m_min <= 0.5)
        def _slow():
            for c in range(n_chunks):
                r0 = c * R
                cp_x = pltpu.async_copy(
                    x_hbm.at[pl.ds(base + r0, R)], xbufs[0], sems_x[0])
                cp_pe = pltpu.async_copy(
                    pe_hbm.at[idx_v.at[pl.ds(r0, R)]], pbufs[0], sems_p[0])
                cp_x.wait()
                cp_pe.wait()

                def row_body(r, carry):
                    m = mask_v[pl.ds(r0 + r, L)][0]
                    for j in range(D // L):
                        sl = pl.ds(j * L, L)
                        plsc.addupdate(xbufs[0].at[r, sl], pbufs[0][r, sl] * m)
                    return carry

                lax.fori_loop(0, R, row_body, 0)
                pltpu.sync_copy(xbufs[0], out_hbm.at[pl.ds(base + r0, R)])

    return k


def kernel(x, positions, token_mask, pe):
    B, S, D = x.shape
    N = B * S
    xf = x.reshape(N, D)
    posf = positions.reshape(N)
    maskf = token_mask.reshape(N).astype(jnp.float32)
    rows_per_w = N // NW
    out = _pe_add_kernel(N, D, rows_per_w, R=16)(xf, posf, maskf, pe)
    return out.reshape(B, S, D)


# speedup vs baseline: 1.0126x; 1.0126x over previous
import functools

import jax
import jax.numpy as jnp
from jax import lax
from jax.experimental import pallas as pl
from jax.experimental.pallas import tpu as pltpu
from jax.experimental.pallas import tpu_sc as plsc

L = 16
NC = 2
NS = 16
NW = NC * NS
NBX = 3
NBP = 4
LOOKAHEAD = 2


def _pe_add_kernel(N, D, rows_per_w, R):
    n_chunks = rows_per_w // R
    mesh = plsc.VectorSubcoreMesh(core_axis_name="c", subcore_axis_name="s")

    buf_types = [pltpu.VMEM((R, D), jnp.float32) for _ in range(NBX + NBP)]
    sem_types = [pltpu.SemaphoreType.DMA for _ in range(NBX + 2 * NBP)]

    @functools.partial(
        pl.kernel,
        out_type=jax.ShapeDtypeStruct((N, D), jnp.float32),
        mesh=mesh,
        scratch_types=[
            pltpu.VMEM((rows_per_w,), jnp.int32),
            pltpu.VMEM((rows_per_w + L,), jnp.float32),
        ] + buf_types + sem_types,
    )
    def k(x_hbm, pos_hbm, mask_hbm, pe_hbm, out_hbm, idx_v, mask_v, *rest):
        xbufs = rest[0:NBX]
        pbufs = rest[NBX:NBX + NBP]
        off = NBX + NBP
        sems_x = rest[off:off + NBX]
        sems_p = rest[off + NBX:off + NBX + NBP]
        sems_o = rest[off + NBX + NBP:off + NBX + 2 * NBP]

        wid = lax.axis_index("s") * NC + lax.axis_index("c")
        base = wid * rows_per_w
        pltpu.sync_copy(pos_hbm.at[pl.ds(base, rows_per_w)], idx_v)
        pltpu.sync_copy(mask_hbm.at[pl.ds(base, rows_per_w)],
                        mask_v.at[pl.ds(0, rows_per_w)])

        def min_body(i, acc):
            return jnp.minimum(acc, mask_v[pl.ds(i * L, L)])

        acc = lax.fori_loop(0, rows_per_w // L, min_body,
                            jnp.full((L,), 1.0, jnp.float32))
        m_min = acc[0]
        for i in range(1, L):
            m_min = jnp.minimum(m_min, acc[i])

        def start_x(c):
            return pltpu.async_copy(
                x_hbm.at[pl.ds(base + c * R, R)], xbufs[c % NBX],
                sems_x[c % NBX])

        def start_pe(c):
            return pltpu.async_copy(
                pe_hbm.at[idx_v.at[pl.ds(c * R, R)]], pbufs[c % NBP],
                sems_p[c % NBP])

        @pl.when(m_min > 0.5)
        def _fast():
            cp_x = [None] * NBX
            cp_p = [None] * NBP
            cp_o = [None] * NBP
            for c in range(min(LOOKAHEAD, n_chunks)):
                cp_x[c % NBX] = start_x(c)
                cp_p[c % NBP] = start_pe(c)
            for c in range(n_chunks):
                sx = c % NBX
                sp = c % NBP
                if c + LOOKAHEAD < n_chunks:
                    c2 = c + LOOKAHEAD
                    if cp_o[c2 % NBP] is not None:
                        cp_o[c2 % NBP].wait()
                    cp_x[c2 % NBX] = start_x(c2)
                    cp_p[c2 % NBP] = start_pe(c2)
                cp_x[sx].wait()
                cp_p[sp].wait()
                xb = xbufs[sx]
                pb = pbufs[sp]

                def combine(r, carry):
                    for j in range(D // L):
                        sl = pl.ds(j * L, L)
                        plsc.addupdate(pb.at[r, sl], xb[r, sl])
                    return carry

                lax.fori_loop(0, R, combine, 0)
                cp_o[sp] = pltpu.async_copy(
                    pb, out_hbm.at[pl.ds(base + c * R, R)], sems_o[sp])
            for s in range(NBP):
                if cp_o[s] is not None:
                    cp_o[s].wait()

        @pl.when(m_min <= 0.5)
        def _slow():
            for c in range(n_chunks):
                r0 = c * R
                cp_x = pltpu.async_copy(
                    x_hbm.at[pl.ds(base + r0, R)], xbufs[0], sems_x[0])
                cp_pe = pltpu.async_copy(
                    pe_hbm.at[idx_v.at[pl.ds(r0, R)]], pbufs[0], sems_p[0])
                cp_x.wait()
                cp_pe.wait()

                def row_body(r, carry):
                    m = mask_v[pl.ds(r0 + r, L)][0]
                    for j in range(D // L):
                        sl = pl.ds(j * L, L)
                        pbufs[0][r, sl] = (pbufs[0][r, sl] * m
                                           + xbufs[0][r, sl])
                    return carry

                lax.fori_loop(0, R, row_body, 0)
                pltpu.sync_copy(pbufs[0], out_hbm.at[pl.ds(base + r0, R)])

    return k


def kernel(x, positions, token_mask, pe):
    B, S, D = x.shape
    N = B * S
    xf = x.reshape(N, D)
    posf = positions.reshape(N)
    maskf = token_mask.reshape(N).astype(jnp.float32)
    rows_per_w = N // NW
    out = _pe_add_kernel(N, D, rows_per_w, R=16)(xf, posf, maskf, pe)
    return out.reshape(B, S, D)


# speedup vs baseline: 1.0160x; 1.0033x over previous
import functools

import jax
import jax.numpy as jnp
from jax import lax
from jax.experimental import pallas as pl
from jax.experimental.pallas import tpu as pltpu
from jax.experimental.pallas import tpu_sc as plsc

L = 16
NC = 2
NS = 16
NW = NC * NS
NBX = 3
NBP = 4
LOOKAHEAD = 2


def _pe_add_kernel(N, D, rows_per_w, R):
    n_chunks = rows_per_w // R
    mesh = plsc.VectorSubcoreMesh(core_axis_name="c", subcore_axis_name="s")

    buf_types = [pltpu.VMEM((R, D), jnp.float32) for _ in range(NBX + NBP)]
    sem_types = [pltpu.SemaphoreType.DMA for _ in range(NBX + 2 * NBP)]

    @functools.partial(
        pl.kernel,
        out_type=jax.ShapeDtypeStruct((N, D), jnp.float32),
        mesh=mesh,
        scratch_types=[
            pltpu.VMEM((rows_per_w,), jnp.int32),
            pltpu.VMEM((rows_per_w + L,), jnp.float32),
        ] + buf_types + sem_types,
    )
    def k(x_hbm, pos_hbm, mask_hbm, pe_hbm, out_hbm, idx_v, mask_v, *rest):
        xbufs = rest[0:NBX]
        pbufs = rest[NBX:NBX + NBP]
        off = NBX + NBP
        sems_x = rest[off:off + NBX]
        sems_p = rest[off + NBX:off + NBX + NBP]
        sems_o = rest[off + NBX + NBP:off + NBX + 2 * NBP]

        wid = lax.axis_index("s") * NC + lax.axis_index("c")
        base = wid * rows_per_w

        def start_x(c):
            return pltpu.async_copy(
                x_hbm.at[pl.ds(base + c * R, R)], xbufs[c % NBX],
                sems_x[c % NBX])

        def start_pe(c):
            return pltpu.async_copy(
                pe_hbm.at[idx_v.at[pl.ds(c * R, R)]], pbufs[c % NBP],
                sems_p[c % NBP])

        pre_x = [start_x(c) for c in range(min(LOOKAHEAD, n_chunks))]
        cp_idx = pltpu.async_copy(
            pos_hbm.at[pl.ds(base, rows_per_w)], idx_v, sems_o[0])
        cp_msk = pltpu.async_copy(
            mask_hbm.at[pl.ds(base, rows_per_w)],
            mask_v.at[pl.ds(0, rows_per_w)], sems_o[1])
        cp_idx.wait()
        pre_p = [start_pe(c) for c in range(min(LOOKAHEAD, n_chunks))]
        cp_msk.wait()

        def min_body(i, acc):
            return jnp.minimum(acc, mask_v[pl.ds(i * L, L)])

        acc = lax.fori_loop(0, rows_per_w // L, min_body,
                            jnp.full((L,), 1.0, jnp.float32))
        m_min = acc[0]
        for i in range(1, L):
            m_min = jnp.minimum(m_min, acc[i])

        @pl.when(m_min > 0.5)
        def _fast():
            cp_x = [None] * NBX
            cp_p = [None] * NBP
            cp_o = [None] * NBP
            for c in range(min(LOOKAHEAD, n_chunks)):
                cp_x[c % NBX] = pre_x[c]
                cp_p[c % NBP] = pre_p[c]
            for c in range(n_chunks):
                sx = c % NBX
                sp = c % NBP
                if c + LOOKAHEAD < n_chunks:
                    c2 = c + LOOKAHEAD
                    if cp_o[c2 % NBP] is not None:
                        cp_o[c2 % NBP].wait()
                    cp_x[c2 % NBX] = start_x(c2)
                    cp_p[c2 % NBP] = start_pe(c2)
                cp_x[sx].wait()
                cp_p[sp].wait()
                xb = xbufs[sx]
                pb = pbufs[sp]

                def combine(r, carry):
                    for j in range(D // L):
                        sl = pl.ds(j * L, L)
                        plsc.addupdate(pb.at[r, sl], xb[r, sl])
                    return carry

                lax.fori_loop(0, R, combine, 0)
                cp_o[sp] = pltpu.async_copy(
                    pb, out_hbm.at[pl.ds(base + c * R, R)], sems_o[sp])
            for s in range(NBP):
                if cp_o[s] is not None:
                    cp_o[s].wait()

        @pl.when(m_min <= 0.5)
        def _slow():
            for cp in pre_x + pre_p:
                cp.wait()
            for c in range(n_chunks):
                r0 = c * R
                cp_x = pltpu.async_copy(
                    x_hbm.at[pl.ds(base + r0, R)], xbufs[0], sems_x[0])
                cp_pe = pltpu.async_copy(
                    pe_hbm.at[idx_v.at[pl.ds(r0, R)]], pbufs[0], sems_p[0])
                cp_x.wait()
                cp_pe.wait()

                def row_body(r, carry):
                    m = mask_v[pl.ds(r0 + r, L)][0]
                    for j in range(D // L):
                        sl = pl.ds(j * L, L)
                        pbufs[0][r, sl] = (pbufs[0][r, sl] * m
                                           + xbufs[0][r, sl])
                    return carry

                lax.fori_loop(0, R, row_body, 0)
                pltpu.sync_copy(pbufs[0], out_hbm.at[pl.ds(base + r0, R)])

    return k


def kernel(x, positions, token_mask, pe):
    B, S, D = x.shape
    N = B * S
    xf = x.reshape(N, D)
    posf = positions.reshape(N)
    maskf = token_mask.reshape(N).astype(jnp.float32)
    rows_per_w = N // NW
    out = _pe_add_kernel(N, D, rows_per_w, R=16)(xf, posf, maskf, pe)
    return out.reshape(B, S, D)


# speedup vs baseline: 1.0182x; 1.0022x over previous
import functools

import jax
import jax.numpy as jnp
from jax import lax
from jax.experimental import pallas as pl
from jax.experimental.pallas import tpu as pltpu
from jax.experimental.pallas import tpu_sc as plsc

L = 16
NC = 2
NS = 16
NW = NC * NS
NBX = 3
NBP = 4
LOOKAHEAD = 2


def _pe_add_kernel(N, D, rows_per_w, R):
    n_chunks = rows_per_w // R
    mesh = plsc.VectorSubcoreMesh(core_axis_name="c", subcore_axis_name="s")

    buf_types = [pltpu.VMEM((R, D), jnp.float32) for _ in range(NBX + NBP)]
    sem_types = [pltpu.SemaphoreType.DMA for _ in range(NBX + 2 * NBP)]

    @functools.partial(
        pl.kernel,
        out_type=jax.ShapeDtypeStruct((N, D), jnp.float32),
        mesh=mesh,
        scratch_types=[
            pltpu.VMEM((rows_per_w,), jnp.int32),
            pltpu.VMEM((rows_per_w + L,), jnp.float32),
        ] + buf_types + sem_types,
    )
    def k(x_hbm, pos_hbm, mask_hbm, pe_hbm, out_hbm, idx_v, mask_v, *rest):
        xbufs = rest[0:NBX]
        pbufs = rest[NBX:NBX + NBP]
        off = NBX + NBP
        sems_x = rest[off:off + NBX]
        sems_p = rest[off + NBX:off + NBX + NBP]
        sems_o = rest[off + NBX + NBP:off + NBX + 2 * NBP]

        wid = lax.axis_index("s") * NC + lax.axis_index("c")
        base = wid * rows_per_w

        def start_x(c):
            return pltpu.async_copy(
                x_hbm.at[pl.ds(base + c * R, R)], xbufs[c % NBX],
                sems_x[c % NBX])

        def start_pe(c):
            return pltpu.async_copy(
                pe_hbm.at[idx_v.at[pl.ds(c * R, R)]], pbufs[c % NBP],
                sems_p[c % NBP])

        pre_x = [start_x(c) for c in range(min(LOOKAHEAD, n_chunks))]
        cp_idx = pltpu.async_copy(
            pos_hbm.at[pl.ds(base, rows_per_w)], idx_v, sems_o[0])
        cp_msk = pltpu.async_copy(
            mask_hbm.at[pl.ds(base, rows_per_w)],
            mask_v.at[pl.ds(0, rows_per_w)], sems_o[1])
        cp_idx.wait()
        pre_p = [start_pe(c) for c in range(min(LOOKAHEAD, n_chunks))]
        cp_msk.wait()

        def min_body(i, acc):
            return jnp.minimum(acc, mask_v[pl.ds(i * L, L)])

        acc = lax.fori_loop(0, rows_per_w // L, min_body,
                            jnp.full((L,), 1.0, jnp.float32))
        m_min = acc[0]
        for i in range(1, L):
            m_min = jnp.minimum(m_min, acc[i])

        @pl.when(m_min > 0.5)
        def _fast():
            cp_x = [None] * NBX
            cp_p = [None] * NBP
            cp_o = [None] * NBP
            for c in range(min(LOOKAHEAD, n_chunks)):
                cp_x[c % NBX] = pre_x[c]
                cp_p[c % NBP] = pre_p[c]
            for c in range(n_chunks):
                sx = c % NBX
                sp = c % NBP
                if c + LOOKAHEAD < n_chunks:
                    c2 = c + LOOKAHEAD
                    if cp_o[c2 % NBP] is not None:
                        cp_o[c2 % NBP].wait()
                    cp_x[c2 % NBX] = start_x(c2)
                    cp_p[c2 % NBP] = start_pe(c2)
                cp_x[sx].wait()
                cp_p[sp].wait()
                xb = xbufs[sx]
                pb = pbufs[sp]

                def combine(r, carry):
                    for j in range(D // L):
                        sl = pl.ds(j * L, L)
                        plsc.addupdate(pb.at[r, sl], xb[r, sl])
                    return carry

                lax.fori_loop(0, R, combine, 0, unroll=2)
                cp_o[sp] = pltpu.async_copy(
                    pb, out_hbm.at[pl.ds(base + c * R, R)], sems_o[sp])
            for s in range(NBP):
                if cp_o[s] is not None:
                    cp_o[s].wait()

        @pl.when(m_min <= 0.5)
        def _slow():
            for cp in pre_x + pre_p:
                cp.wait()

            def chunk_body(c, carry):
                r0 = c * R
                cp_x = pltpu.async_copy(
                    x_hbm.at[pl.ds(base + r0, R)], xbufs[0], sems_x[0])
                cp_pe = pltpu.async_copy(
                    pe_hbm.at[idx_v.at[pl.ds(r0, R)]], pbufs[0], sems_p[0])
                cp_x.wait()
                cp_pe.wait()

                def row_body(r, rcarry):
                    m = mask_v[pl.ds(r0 + r, L)][0]
                    for j in range(D // L):
                        sl = pl.ds(j * L, L)
                        pbufs[0][r, sl] = (pbufs[0][r, sl] * m
                                           + xbufs[0][r, sl])
                    return rcarry

                lax.fori_loop(0, R, row_body, 0)
                pltpu.sync_copy(pbufs[0], out_hbm.at[pl.ds(base + r0, R)])
                return carry

            lax.fori_loop(0, n_chunks, chunk_body, 0)

    return k


def kernel(x, positions, token_mask, pe):
    B, S, D = x.shape
    N = B * S
    xf = x.reshape(N, D)
    posf = positions.reshape(N)
    maskf = token_mask.reshape(N).astype(jnp.float32)
    rows_per_w = N // NW
    out = _pe_add_kernel(N, D, rows_per_w, R=16)(xf, posf, maskf, pe)
    return out.reshape(B, S, D)


# speedup vs baseline: 1.0246x; 1.0064x over previous
import functools

import jax
import jax.numpy as jnp
from jax import lax
from jax.experimental import pallas as pl
from jax.experimental.pallas import tpu as pltpu
from jax.experimental.pallas import tpu_sc as plsc

L = 16
NC = 2
NS = 16
NW = NC * NS
NBX = 3
NBP = 4
LOOKAHEAD = 2


def _pe_add_kernel(N, D, S, rows_per_w, R):
    n_chunks = rows_per_w // R
    w_per_row = S // rows_per_w
    mesh = plsc.VectorSubcoreMesh(core_axis_name="c", subcore_axis_name="s")

    buf_types = [pltpu.VMEM((R, D), jnp.float32) for _ in range(NBX + NBP)]
    sem_types = [pltpu.SemaphoreType.DMA for _ in range(NBX + 2 * NBP)]

    @functools.partial(
        pl.kernel,
        out_type=jax.ShapeDtypeStruct((N, D), jnp.float32),
        mesh=mesh,
        scratch_types=[
            pltpu.VMEM((rows_per_w,), jnp.int32),
            pltpu.VMEM((rows_per_w + L,), jnp.float32),
        ] + buf_types + sem_types,
    )
    def k(x_hbm, pos_hbm, mask_hbm, pe_hbm, out_hbm, idx_v, mask_v, *rest):
        xbufs = rest[0:NBX]
        pbufs = rest[NBX:NBX + NBP]
        off = NBX + NBP
        sems_x = rest[off:off + NBX]
        sems_p = rest[off + NBX:off + NBX + NBP]
        sems_o = rest[off + NBX + NBP:off + NBX + 2 * NBP]

        wid = lax.axis_index("s") * NC + lax.axis_index("c")
        base = wid * rows_per_w
        bi = wid // w_per_row
        s0 = (wid % w_per_row) * rows_per_w

        def start_x(c):
            return pltpu.async_copy(
                x_hbm.at[pl.ds(base + c * R, R)], xbufs[c % NBX],
                sems_x[c % NBX])

        def start_pe(c):
            return pltpu.async_copy(
                pe_hbm.at[idx_v.at[pl.ds(c * R, R)]], pbufs[c % NBP],
                sems_p[c % NBP])

        pre_x = [start_x(c) for c in range(min(LOOKAHEAD, n_chunks))]
        cp_idx = pltpu.async_copy(
            pos_hbm.at[bi, pl.ds(s0, rows_per_w)], idx_v, sems_o[0])
        cp_msk = pltpu.async_copy(
            mask_hbm.at[bi, pl.ds(s0, rows_per_w)],
            mask_v.at[pl.ds(0, rows_per_w)], sems_o[1])
        cp_idx.wait()
        pre_p = [start_pe(c) for c in range(min(LOOKAHEAD, n_chunks))]
        cp_msk.wait()

        def min_body(i, acc):
            return jnp.minimum(acc, mask_v[pl.ds(i * L, L)])

        acc = lax.fori_loop(0, rows_per_w // L, min_body,
                            jnp.full((L,), 1.0, jnp.float32))
        m_min = acc[0]
        for i in range(1, L):
            m_min = jnp.minimum(m_min, acc[i])

        @pl.when(m_min > 0.5)
        def _fast():
            cp_x = [None] * NBX
            cp_p = [None] * NBP
            cp_o = [None] * NBP
            for c in range(min(LOOKAHEAD, n_chunks)):
                cp_x[c % NBX] = pre_x[c]
                cp_p[c % NBP] = pre_p[c]
            for c in range(n_chunks):
                sx = c % NBX
                sp = c % NBP
                if c + LOOKAHEAD < n_chunks:
                    c2 = c + LOOKAHEAD
                    if cp_o[c2 % NBP] is not None:
                        cp_o[c2 % NBP].wait()
                    cp_x[c2 % NBX] = start_x(c2)
                    cp_p[c2 % NBP] = start_pe(c2)
                cp_x[sx].wait()
                cp_p[sp].wait()
                xb = xbufs[sx]
                pb = pbufs[sp]

                def combine(r, carry):
                    for j in range(D // L):
                        sl = pl.ds(j * L, L)
                        plsc.addupdate(pb.at[r, sl], xb[r, sl])
                    return carry

                lax.fori_loop(0, R, combine, 0, unroll=2)
                cp_o[sp] = pltpu.async_copy(
                    pb, out_hbm.at[pl.ds(base + c * R, R)], sems_o[sp])
            for s in range(NBP):
                if cp_o[s] is not None:
                    cp_o[s].wait()

        @pl.when(m_min <= 0.5)
        def _slow():
            for cp in pre_x + pre_p:
                cp.wait()

            def chunk_body(c, carry):
                r0 = c * R
                cp_x = pltpu.async_copy(
                    x_hbm.at[pl.ds(base + r0, R)], xbufs[0], sems_x[0])
                cp_pe = pltpu.async_copy(
                    pe_hbm.at[idx_v.at[pl.ds(r0, R)]], pbufs[0], sems_p[0])
                cp_x.wait()
                cp_pe.wait()

                def row_body(r, rcarry):
                    m = mask_v[pl.ds(r0 + r, L)][0]
                    for j in range(D // L):
                        sl = pl.ds(j * L, L)
                        pbufs[0][r, sl] = (pbufs[0][r, sl] * m
                                           + xbufs[0][r, sl])
                    return rcarry

                lax.fori_loop(0, R, row_body, 0)
                pltpu.sync_copy(pbufs[0], out_hbm.at[pl.ds(base + r0, R)])
                return carry

            lax.fori_loop(0, n_chunks, chunk_body, 0)

    return k


def kernel(x, positions, token_mask, pe):
    B, S, D = x.shape
    N = B * S
    xf = x.reshape(N, D)
    maskf = token_mask.astype(jnp.float32)
    rows_per_w = N // NW
    out = _pe_add_kernel(N, D, S, rows_per_w, R=16)(xf, positions, maskf, pe)
    return out.reshape(B, S, D)
